# Initial kernel scaffold; baseline (speedup 1.0000x reference)
import jax
import jax.numpy as jnp
from jax.experimental import pallas as pl


def kernel(hidden_states, wq, wk, wv, wo, pre_attn_norm_w, post_attn_norm_w, pre_moe_norm_w, post_moe_norm_w, router_w, expert_w_gate, expert_w_lin, expert_w_out, position_ids):
    raise NotImplementedError("write your pallas kernel here")


# speedup vs baseline: 1.8345x; 1.8345x over previous
import functools

import jax
import jax.numpy as jnp
import numpy as np
from jax import lax
from jax.experimental import pallas as pl
from jax.experimental.pallas import tpu as pltpu
from jax.experimental.pallas import tpu_sc as plsc

S = 2048
H = 1024
NH, KVH, HD = 16, 8, 64
E, TOPK, FF = 8, 2, 4096
EPS = 1e-5
PREC = lax.Precision.HIGHEST
PREC_ATT = lax.Precision.DEFAULT
BF16 = jnp.bfloat16
MAX_ATTN = 30.0
ATTN_MULT = 0.125
ROPE_THETA = 100000.0

NPAIR = S * TOPK
BLK = 512
NBLK = 16
CAP = NBLK * BLK
FBLK = 512
NF = FF // FBLK

QB = 512
RB = 512
NR = S // RB
F32 = jnp.float32

SC_NC, SC_NS = 2, 16
SC_NW = SC_NC * SC_NS
PAIRS_PER_W = NPAIR // SC_NW
SC_CHUNK = 64

def _qkv_kernel(hs_ref, wq_ref, wk_ref, wv_ref, nw_ref, pos_ref,
                q_ref, k_ref, v_ref):
    h = hs_ref[...]
    var = jnp.mean(h * h, axis=1, keepdims=True)
    hn = h * lax.rsqrt(var + EPS) * nw_ref[...]
    q = jnp.dot(hn, wq_ref[...], preferred_element_type=F32,
                precision=PREC_ATT)
    k = jnp.dot(hn, wk_ref[...], preferred_element_type=F32,
                precision=PREC_ATT)
    v = jnp.dot(hn, wv_ref[...], preferred_element_type=F32,
                precision=PREC_ATT)
    pos = pos_ref[...].astype(F32)
    dim = lax.broadcasted_iota(jnp.int32, (1, HD // 2), 1).astype(F32) * 2.0
    inv_freq = jnp.exp(dim * np.float32(-np.log(ROPE_THETA) / HD))
    freqs = pos * inv_freq
    cos = jnp.cos(freqs)
    sin = jnp.sin(freqs)
    hf = HD // 2
    for hh in range(NH):
        qh = q[:, hh * HD:(hh + 1) * HD]
        q1, q2 = qh[:, :hf], qh[:, hf:]
        q_ref[hh] = jnp.concatenate(
            [q1 * cos - q2 * sin, q2 * cos + q1 * sin], axis=1)
    for hh in range(KVH):
        kh = k[:, hh * HD:(hh + 1) * HD]
        k1, k2 = kh[:, :hf], kh[:, hf:]
        k_ref[hh] = jnp.concatenate(
            [k1 * cos - k2 * sin, k2 * cos + k1 * sin], axis=1)
        v_ref[hh] = v[:, hh * HD:(hh + 1) * HD]


def _qkv_call(hs, wq, wk, wv, nw, pos):
    return pl.pallas_call(
        _qkv_kernel,
        grid=(NR,),
        in_specs=[
            pl.BlockSpec((RB, H), lambda i: (i, 0)),
            pl.BlockSpec((H, NH * HD), lambda i: (0, 0)),
            pl.BlockSpec((H, KVH * HD), lambda i: (0, 0)),
            pl.BlockSpec((H, KVH * HD), lambda i: (0, 0)),
            pl.BlockSpec((1, H), lambda i: (0, 0)),
            pl.BlockSpec((RB, 1), lambda i: (i, 0)),
        ],
        out_specs=[
            pl.BlockSpec((NH, RB, HD), lambda i: (0, i, 0)),
            pl.BlockSpec((KVH, RB, HD), lambda i: (0, i, 0)),
            pl.BlockSpec((KVH, RB, HD), lambda i: (0, i, 0)),
        ],
        out_shape=[jax.ShapeDtypeStruct((NH, S, HD), F32),
                   jax.ShapeDtypeStruct((KVH, S, HD), F32),
                   jax.ShapeDtypeStruct((KVH, S, HD), F32)],
    )(hs, wq, wk, wv, nw, pos)


def _attn_kernel(q_ref, k_ref, v_ref, o_ref):
    q = q_ref[0]
    k = k_ref[0]
    s = lax.dot_general(q, k, (((1,), (1,)), ((), ())),
                        preferred_element_type=F32,
                        precision=PREC_ATT) * ATTN_MULT
    s = MAX_ATTN * jnp.tanh(s * (1.0 / MAX_ATTN))
    m = jnp.max(s, axis=1, keepdims=True)
    p = jnp.exp(s - m)
    p = p / jnp.sum(p, axis=1, keepdims=True)
    o_ref[0] = jnp.dot(p, v_ref[0], preferred_element_type=F32,
                       precision=PREC_ATT)


def _post_attn_kernel(ctx_ref, wo_ref, hs_ref, postw_ref, premoew_ref,
                      rw_ref, hid_ref, x_ref, w1_ref, w2_ref, i1_ref,
                      i2_ref):
    ctx2d = jnp.concatenate([ctx_ref[hh] for hh in range(NH)], axis=1)
    ctx2d = ctx2d.astype(BF16).astype(F32)
    attn = jnp.dot(ctx2d, wo_ref[...], preferred_element_type=F32,
                   precision=PREC)
    var = jnp.mean(attn * attn, axis=1, keepdims=True)
    hid = hs_ref[...] + attn * lax.rsqrt(var + EPS) * postw_ref[...]
    hid_ref[...] = hid
    var2 = jnp.mean(hid * hid, axis=1, keepdims=True)
    x = hid * lax.rsqrt(var2 + EPS) * premoew_ref[...]
    x_ref[...] = x

    xb = x.astype(BF16).astype(F32)
    logits = jnp.dot(xb, rw_ref[...], preferred_element_type=F32,
                     precision=PREC)
    colid = lax.broadcasted_iota(jnp.int32, (RB, E), 1)
    m1 = jnp.max(logits, axis=1, keepdims=True)
    i1 = jnp.min(jnp.where(logits == m1, colid, E), axis=1, keepdims=True)
    l2 = jnp.where(colid == i1, -jnp.inf, logits)
    m2 = jnp.max(l2, axis=1, keepdims=True)
    i2 = jnp.min(jnp.where(l2 == m2, colid, E), axis=1, keepdims=True)
    w1 = 1.0 / (1.0 + jnp.exp(m2 - m1))
    w1_ref[...] = w1
    w2_ref[...] = 1.0 - w1
    i1_ref[...] = i1
    i2_ref[...] = i2


def _route_kernel(i1_ref, i2_ref, dest_ref, meta_ref):
    ep = jnp.concatenate([i1_ref[...], i2_ref[...]], axis=0)
    cols = lax.broadcasted_iota(jnp.int32, (NPAIR, E), 1)
    onehot = (ep == cols).astype(jnp.int32)
    inc = onehot
    sh = 1
    while sh < NPAIR:
        shifted = jnp.concatenate(
            [jnp.zeros((sh, E), jnp.int32), inc[:NPAIR - sh]], axis=0)
        inc = inc + shifted
        sh *= 2
    counts = inc[NPAIR - 1:NPAIR, :]
    nblk = (counts + (BLK - 1)) // BLK
    cb = nblk
    sh = 1
    while sh < E:
        cb = cb + jnp.concatenate(
            [jnp.zeros((1, sh), jnp.int32), cb[:, :E - sh]], axis=1)
        sh *= 2
    excl_blk = cb - nblk
    row_off = excl_blk * BLK
    rank = jnp.sum(onehot * (inc - 1), axis=1, keepdims=True)
    base = jnp.sum(onehot * row_off, axis=1, keepdims=True)
    dest_ref[...] = base + rank

    total = jnp.sum(nblk, axis=1, keepdims=True)
    bid = lax.broadcasted_iota(jnp.int32, (NBLK, E), 0)
    ge = (bid >= excl_blk).astype(jnp.int32)
    be = jnp.sum(ge, axis=1, keepdims=True) - 1
    bvalid = lax.broadcasted_iota(jnp.int32, (NBLK, 1), 0) < total
    meta_ref[...] = jnp.where(bvalid, be, E)


def _dispatch_body(x_hbm, dest_hbm, xs_hbm, idx_v, rows_v, sem):
    wid = lax.axis_index("s") * SC_NC + lax.axis_index("c")
    for c in range(PAIRS_PER_W // SC_CHUNK):
        base = wid * PAIRS_PER_W + c * SC_CHUNK
        tok = lax.rem(base, S)
        pltpu.sync_copy(dest_hbm.at[pl.ds(base, SC_CHUNK)], idx_v)
        pltpu.sync_copy(x_hbm.at[pl.ds(tok, SC_CHUNK)], rows_v)
        pltpu.async_copy(rows_v, xs_hbm.at[idx_v], sem).wait()


def _combine_body(y_hbm, dest_hbm, yg_hbm, idx_v, rows_v, sem):
    wid = lax.axis_index("s") * SC_NC + lax.axis_index("c")
    for c in range(PAIRS_PER_W // SC_CHUNK):
        base = wid * PAIRS_PER_W + c * SC_CHUNK
        pltpu.sync_copy(dest_hbm.at[pl.ds(base, SC_CHUNK)], idx_v)
        pltpu.async_copy(y_hbm.at[idx_v], rows_v, sem).wait()
        pltpu.sync_copy(rows_v, yg_hbm.at[pl.ds(base, SC_CHUNK)])


def _ffn_kernel(meta_ref, xs_ref, wg_ref, wl_ref, wo_ref, y_ref):
    b = pl.program_id(0)
    f = pl.program_id(1)
    e = meta_ref[b]

    @pl.when(e < E)
    def _():
        x = xs_ref[...]
        g = jnp.dot(x, wg_ref[0], preferred_element_type=F32,
                    precision=PREC_ATT)
        ge = 0.5 * g * (1.0 + lax.erf(g * np.float32(1.0 / np.sqrt(2.0))))
        vv = jnp.dot(x, wl_ref[0], preferred_element_type=F32,
                     precision=PREC_ATT)
        contrib = jnp.dot(ge * vv, wo_ref[0],
                          preferred_element_type=F32, precision=PREC)

        @pl.when(f == 0)
        def _():
            y_ref[...] = contrib

        @pl.when(f > 0)
        def _():
            y_ref[...] += contrib


def _final_kernel(yg_ref, w1_ref, w2_ref, hid_ref, pw_ref, out_ref):
    moe = w1_ref[...] * yg_ref[0] + w2_ref[...] * yg_ref[1]
    var = jnp.mean(moe * moe, axis=1, keepdims=True)
    out_ref[...] = hid_ref[...] + moe * lax.rsqrt(var + EPS) * pw_ref[...]


def kernel(hidden_states, wq, wk, wv, wo, pre_attn_norm_w, post_attn_norm_w,
           pre_moe_norm_w, post_moe_norm_w, router_w, expert_w_gate,
           expert_w_lin, expert_w_out, position_ids):
    hs = hidden_states.reshape(S, H)
    pos = position_ids.reshape(S, 1)

    q, k, v = _qkv_call(hs, wq, wk, wv, pre_attn_norm_w.reshape(1, H), pos)

    ctx = pl.pallas_call(
        _attn_kernel,
        grid=(NH, S // QB),
        in_specs=[
            pl.BlockSpec((1, QB, HD), lambda h, i: (h, i, 0)),
            pl.BlockSpec((1, S, HD), lambda h, i: (h // (NH // KVH), 0, 0)),
            pl.BlockSpec((1, S, HD), lambda h, i: (h // (NH // KVH), 0, 0)),
        ],
        out_specs=pl.BlockSpec((1, QB, HD), lambda h, i: (h, i, 0)),
        out_shape=jax.ShapeDtypeStruct((NH, S, HD), F32),
    )(q, k, v)

    hid, x, w1, w2, i1, i2 = pl.pallas_call(
        _post_attn_kernel,
        grid=(NR,),
        in_specs=[
            pl.BlockSpec((NH, RB, HD), lambda i: (0, i, 0)),
            pl.BlockSpec((NH * HD, H), lambda i: (0, 0)),
            pl.BlockSpec((RB, H), lambda i: (i, 0)),
            pl.BlockSpec((1, H), lambda i: (0, 0)),
            pl.BlockSpec((1, H), lambda i: (0, 0)),
            pl.BlockSpec((H, E), lambda i: (0, 0)),
        ],
        out_specs=[
            pl.BlockSpec((RB, H), lambda i: (i, 0)),
            pl.BlockSpec((RB, H), lambda i: (i, 0)),
            pl.BlockSpec((RB, 1), lambda i: (i, 0)),
            pl.BlockSpec((RB, 1), lambda i: (i, 0)),
            pl.BlockSpec((RB, 1), lambda i: (i, 0)),
            pl.BlockSpec((RB, 1), lambda i: (i, 0)),
        ],
        out_shape=[jax.ShapeDtypeStruct((S, H), F32),
                   jax.ShapeDtypeStruct((S, H), F32),
                   jax.ShapeDtypeStruct((S, 1), F32),
                   jax.ShapeDtypeStruct((S, 1), F32),
                   jax.ShapeDtypeStruct((S, 1), jnp.int32),
                   jax.ShapeDtypeStruct((S, 1), jnp.int32)],
    )(ctx, wo, hs, post_attn_norm_w.reshape(1, H),
      pre_moe_norm_w.reshape(1, H), router_w)

    dest, meta = pl.pallas_call(
        _route_kernel,
        out_shape=[jax.ShapeDtypeStruct((NPAIR, 1), jnp.int32),
                   jax.ShapeDtypeStruct((NBLK, 1), jnp.int32)],
    )(i1, i2)

    dest1d = dest.reshape(NPAIR)
    meta1d = meta.reshape(NBLK)

    sc_mesh = plsc.VectorSubcoreMesh(core_axis_name="c", subcore_axis_name="s")
    xs = pl.kernel(
        _dispatch_body,
        out_type=jax.ShapeDtypeStruct((CAP, H), F32),
        mesh=sc_mesh,
        scratch_types=[pltpu.VMEM((SC_CHUNK,), jnp.int32),
                       pltpu.VMEM((SC_CHUNK, H), F32),
                       pltpu.SemaphoreType.DMA],
    )(x, dest1d)

    y = pl.pallas_call(
        _ffn_kernel,
        grid_spec=pltpu.PrefetchScalarGridSpec(
            num_scalar_prefetch=1,
            grid=(NBLK, NF),
            in_specs=[
                pl.BlockSpec((BLK, H), lambda b, f, m: (b, 0)),
                pl.BlockSpec((1, H, FBLK),
                             lambda b, f, m: (jnp.minimum(m[b], E - 1), 0, f)),
                pl.BlockSpec((1, H, FBLK),
                             lambda b, f, m: (jnp.minimum(m[b], E - 1), 0, f)),
                pl.BlockSpec((1, FBLK, H),
                             lambda b, f, m: (jnp.minimum(m[b], E - 1), f, 0)),
            ],
            out_specs=pl.BlockSpec((BLK, H), lambda b, f, m: (b, 0)),
        ),
        out_shape=jax.ShapeDtypeStruct((CAP, H), F32),
        compiler_params=pltpu.CompilerParams(
            dimension_semantics=("arbitrary", "arbitrary")),
    )(meta1d, xs, expert_w_gate, expert_w_lin, expert_w_out)

    yg = pl.kernel(
        _combine_body,
        out_type=jax.ShapeDtypeStruct((NPAIR, H), F32),
        mesh=sc_mesh,
        scratch_types=[pltpu.VMEM((SC_CHUNK,), jnp.int32),
                       pltpu.VMEM((SC_CHUNK, H), F32),
                       pltpu.SemaphoreType.DMA],
    )(y, dest1d)

    out = pl.pallas_call(
        _final_kernel,
        grid=(S // QB,),
        in_specs=[
            pl.BlockSpec((TOPK, QB, H), lambda i: (0, i, 0)),
            pl.BlockSpec((QB, 1), lambda i: (i, 0)),
            pl.BlockSpec((QB, 1), lambda i: (i, 0)),
            pl.BlockSpec((QB, H), lambda i: (i, 0)),
            pl.BlockSpec((1, H), lambda i: (0, 0)),
        ],
        out_specs=pl.BlockSpec((QB, H), lambda i: (i, 0)),
        out_shape=jax.ShapeDtypeStruct((S, H), F32),
    )(yg.reshape(TOPK, S, H), w1, w2, hid, post_moe_norm_w.reshape(1, H))

    return out.reshape(1, S, H)
